# Initial kernel scaffold; baseline (speedup 1.0000x reference)
import jax
import jax.numpy as jnp
from jax.experimental import pallas as pl


def kernel(input_tokens, inputs, embeddings, bias):
    raise NotImplementedError("write your pallas kernel here")


# speedup vs baseline: 1.4722x; 1.4722x over previous
import functools

import jax
import jax.numpy as jnp
from jax import lax
from jax.experimental import pallas as pl
from jax.experimental.pallas import tpu as pltpu
from jax.experimental.pallas import tpu_sc as plsc

LANES = 16
NC = 2
NS = 16
NW = NC * NS
CHUNK = 128


@functools.lru_cache(maxsize=None)
def _build_sc_kernel(BF: int, D: int, per_w: int):
    n_chunks = per_w // CHUNK
    mesh = plsc.VectorSubcoreMesh(core_axis_name="c", subcore_axis_name="s")

    @functools.partial(
        pl.kernel,
        mesh=mesh,
        out_type=jax.ShapeDtypeStruct((BF, D), jnp.float32),
        compiler_params=pltpu.CompilerParams(use_tc_tiling_on_sc=False),
        scratch_types=[
            pltpu.VMEM((per_w,), jnp.int32),
            pltpu.VMEM((per_w,), jnp.float32),
            pltpu.VMEM((CHUNK, D), jnp.float32),
            pltpu.VMEM((CHUNK, D), jnp.float32),
            pltpu.SemaphoreType.DMA,
        ],
    )
    def sc_kernel(tok_hbm, inp_hbm, emb_hbm, bias_hbm, out_hbm,
                  idx_v, inp_v, emb_v, bias_v, sem):
        wid = lax.axis_index("s") * NC + lax.axis_index("c")
        base = wid * per_w

        pltpu.sync_copy(tok_hbm.at[pl.ds(base, per_w)], idx_v)
        pltpu.sync_copy(inp_hbm.at[pl.ds(base, per_w)], inp_v)

        def chunk_body(c, carry):
            off = c * CHUNK
            idx_slice = idx_v.at[pl.ds(off, CHUNK)]
            ge = pltpu.async_copy(emb_hbm.at[idx_slice], emb_v, sem)
            gb = pltpu.async_copy(bias_hbm.at[idx_slice], bias_v, sem)
            ge.wait()
            gb.wait()

            def group_body(g, carry2):
                row0 = g * LANES
                sv = inp_v[pl.ds(off + row0, LANES)]
                for r in range(LANES):
                    s = sv[r]
                    for h in range(D // LANES):
                        sl = pl.ds(h * LANES, LANES)
                        x = emb_v[row0 + r, sl] * s + bias_v[row0 + r, sl]
                        y = jnp.where(x > 0.0, x, jnp.exp(x) - 1.0)
                        emb_v[row0 + r, sl] = y
                return carry2

            lax.fori_loop(0, CHUNK // LANES, group_body, 0)
            pltpu.sync_copy(emb_v, out_hbm.at[pl.ds(base + off, CHUNK)])
            return carry

        lax.fori_loop(0, n_chunks, chunk_body, 0)

    return sc_kernel


def kernel(input_tokens, inputs, embeddings, bias):
    B, F = input_tokens.shape
    V, D = embeddings.shape
    BF = B * F
    tok = input_tokens.reshape(BF).astype(jnp.int32)
    inp = inputs.reshape(BF).astype(jnp.float32)

    quantum = NW * CHUNK
    BFp = ((BF + quantum - 1) // quantum) * quantum
    if BFp != BF:
        tok = jnp.pad(tok, (0, BFp - BF))
        inp = jnp.pad(inp, (0, BFp - BF))

    out = _build_sc_kernel(BFp, D, BFp // NW)(tok, inp, embeddings, bias)
    if BFp != BF:
        out = out[:BF]
    return out.reshape(B, F, D)


# speedup vs baseline: 1.6275x; 1.1055x over previous
import functools

import jax
import jax.numpy as jnp
from jax import lax
from jax.experimental import pallas as pl
from jax.experimental.pallas import tpu as pltpu
from jax.experimental.pallas import tpu_sc as plsc

LANES = 16
NC = 2
NS = 16
NW = NC * NS
CHUNK = 128
NBUF = 4


@functools.lru_cache(maxsize=None)
def _build_sc_kernel(BF: int, D: int, per_w: int):
    n_chunks = per_w // CHUNK
    assert n_chunks % NBUF == 0
    mesh = plsc.VectorSubcoreMesh(core_axis_name="c", subcore_axis_name="s")

    @functools.partial(
        pl.kernel,
        mesh=mesh,
        out_type=jax.ShapeDtypeStruct((BF, D), jnp.float32),
        compiler_params=pltpu.CompilerParams(use_tc_tiling_on_sc=False),
        scratch_types=(
            [
                pltpu.VMEM((per_w,), jnp.int32),
                pltpu.VMEM((per_w,), jnp.float32),
                pltpu.VMEM((NBUF, CHUNK, D), jnp.float32),
                pltpu.VMEM((NBUF, CHUNK, D), jnp.float32),
                pltpu.VMEM((NBUF, CHUNK, D), jnp.float32),
            ]
            + [pltpu.SemaphoreType.DMA] * (2 * NBUF)
        ),
    )
    def sc_kernel(tok_hbm, inp_hbm, emb_hbm, bias_hbm, out_hbm,
                  idx_v, inp_v, emb_v, bias_v, out_v, *sems):
        g_sem = sems[:NBUF]
        o_sem = sems[NBUF:]
        wid = lax.axis_index("s") * NC + lax.axis_index("c")
        base = wid * per_w

        pltpu.sync_copy(tok_hbm.at[pl.ds(base, per_w)], idx_v)
        pltpu.sync_copy(inp_hbm.at[pl.ds(base, per_w)], inp_v)

        def fire_gathers(c, b):
            idx_slice = idx_v.at[pl.ds(c * CHUNK, CHUNK)]
            pltpu.async_copy(emb_hbm.at[idx_slice], emb_v.at[b], g_sem[b])
            pltpu.async_copy(bias_hbm.at[idx_slice], bias_v.at[b], g_sem[b])

        def wait_gathers(c, b):
            idx_slice = idx_v.at[pl.ds(c * CHUNK, CHUNK)]
            pltpu.make_async_copy(emb_hbm.at[idx_slice], emb_v.at[b],
                                  g_sem[b]).wait()
            pltpu.make_async_copy(bias_hbm.at[idx_slice], bias_v.at[b],
                                  g_sem[b]).wait()

        def out_copy(c, b):
            return pltpu.make_async_copy(
                out_v.at[b], out_hbm.at[pl.ds(base + c * CHUNK, CHUNK)],
                o_sem[b])

        for b in range(NBUF):
            fire_gathers(b, b)

        def ring_body(g, carry):
            for b in range(NBUF):
                c = g * NBUF + b
                wait_gathers(c, b)

                @pl.when(c >= NBUF)
                def _():
                    out_copy(c - NBUF, b).wait()

                def group_body(gr, carry2):
                    row0 = gr * LANES
                    sv = inp_v[pl.ds(c * CHUNK + row0, LANES)]
                    for r in range(LANES):
                        s = sv[r]
                        for h in range(D // LANES):
                            sl = pl.ds(h * LANES, LANES)
                            x = emb_v[b, row0 + r, sl] * s \
                                + bias_v[b, row0 + r, sl]
                            y = jnp.where(x > 0.0, x, jnp.exp(x) - 1.0)
                            out_v[b, row0 + r, sl] = y
                    return carry2

                lax.fori_loop(0, CHUNK // LANES, group_body, 0)
                out_copy(c, b).start()

                @pl.when(c + NBUF < n_chunks)
                def _():
                    fire_gathers(c + NBUF, b)
            return carry

        lax.fori_loop(0, n_chunks // NBUF, ring_body, 0)
        for b in range(NBUF):
            out_copy(n_chunks - NBUF + b, b).wait()

    return sc_kernel


def kernel(input_tokens, inputs, embeddings, bias):
    B, F = input_tokens.shape
    V, D = embeddings.shape
    BF = B * F
    tok = input_tokens.reshape(BF).astype(jnp.int32)
    inp = inputs.reshape(BF).astype(jnp.float32)

    quantum = NW * CHUNK * NBUF
    BFp = ((BF + quantum - 1) // quantum) * quantum
    if BFp != BF:
        tok = jnp.pad(tok, (0, BFp - BF))
        inp = jnp.pad(inp, (0, BFp - BF))

    out = _build_sc_kernel(BFp, D, BFp // NW)(tok, inp, embeddings, bias)
    if BFp != BF:
        out = out[:BF]
    return out.reshape(B, F, D)
